# baseline (speedup 1.0000x reference)
import functools

import jax
import jax.numpy as jnp
from jax import lax
from jax.experimental import pallas as pl
from jax.experimental.pallas import tpu as pltpu
from jax.experimental.pallas import tpu_sc as plsc

L = 16


@jax.jit
def _gather_gps(table_flat, n):
    B = n.shape[0]
    info = plsc.get_sparse_core_info()
    num_workers = info.num_cores * info.num_subcores
    bw = B // num_workers
    mesh = plsc.VectorSubcoreMesh(core_axis_name="c", subcore_axis_name="s")

    @functools.partial(
        pl.kernel,
        mesh=mesh,
        out_type=jax.ShapeDtypeStruct((2 * B,), jnp.float32),
        scratch_types=[
            pltpu.VMEM((bw,), jnp.int32),
            pltpu.VMEM((bw,), jnp.int32),
            pltpu.VMEM((bw,), jnp.int32),
            pltpu.VMEM((bw,), jnp.int32),
            pltpu.VMEM((bw,), jnp.int32),
            pltpu.VMEM((bw,), jnp.float32),
            pltpu.VMEM((bw,), jnp.float32),
            pltpu.SemaphoreType.DMA,
            pltpu.SemaphoreType.DMA,
        ],
    )
    def k(table_hbm, idx_hbm, out_hbm, idx_v, eidx_v, oidx_v, eout_v, oout_v,
          lat_v, lon_v, gsem, ssem):
        wid = lax.axis_index("s") * info.num_cores + lax.axis_index("c")
        base = wid * bw
        pltpu.sync_copy(idx_hbm.at[pl.ds(base, bw)], idx_v)
        lane = lax.iota(jnp.int32, L)
        for j in range(bw // L):
            iv = idx_v[pl.ds(j * L, L)]
            eidx_v[pl.ds(j * L, L)] = iv * 2
            oidx_v[pl.ds(j * L, L)] = iv * 2 + 1
            slot = (base + j * L + lane) * 2
            eout_v[pl.ds(j * L, L)] = slot
            oout_v[pl.ds(j * L, L)] = slot + 1
        h_lat = pltpu.async_copy(table_hbm.at[eidx_v], lat_v, gsem)
        h_lon = pltpu.async_copy(table_hbm.at[oidx_v], lon_v, gsem)
        h_lat.wait()
        h_lon.wait()
        for j in range(bw // L):
            la = lat_v[pl.ds(j * L, L)]
            lat_v[pl.ds(j * L, L)] = jnp.minimum(jnp.maximum(la, -1.0), 1.0) * 90.0
            lo = lon_v[pl.ds(j * L, L)]
            lon_v[pl.ds(j * L, L)] = jnp.minimum(jnp.maximum(lo, -1.0), 1.0) * 180.0
        s_lat = pltpu.async_copy(lat_v, out_hbm.at[eout_v], ssem)
        s_lon = pltpu.async_copy(lon_v, out_hbm.at[oout_v], ssem)
        s_lat.wait()
        s_lon.wait()

    return k(table_flat, n)


def kernel(img, coordinates, n):
    del img
    flat = _gather_gps(coordinates.reshape(-1), n)
    return flat.reshape(n.shape[0], 2)


# speedup vs baseline: 1.5718x; 1.5718x over previous
import functools

import jax
import jax.numpy as jnp
from jax import lax
from jax.experimental import pallas as pl
from jax.experimental.pallas import tpu as pltpu
from jax.experimental.pallas import tpu_sc as plsc

L = 16

_PERM_DNUMS = lax.GatherDimensionNumbers(
    offset_dims=(), collapsed_slice_dims=(0,), start_index_map=(0,))


def _vperm(x, perm):
    return lax.gather(x, perm[:, None], _PERM_DNUMS, slice_sizes=(1,),
                      mode=lax.GatherScatterMode.PROMISE_IN_BOUNDS)


@jax.jit
def _gather_gps(table_flat, n):
    B = n.shape[0]
    info = plsc.get_sparse_core_info()
    num_workers = info.num_cores * info.num_subcores
    bw = B // num_workers
    mesh = plsc.VectorSubcoreMesh(core_axis_name="c", subcore_axis_name="s")

    @functools.partial(
        pl.kernel,
        mesh=mesh,
        out_type=jax.ShapeDtypeStruct((2 * B,), jnp.float32),
        scratch_types=[
            pltpu.VMEM((bw,), jnp.int32),
            pltpu.VMEM((bw,), jnp.int32),
            pltpu.VMEM((bw,), jnp.int32),
            pltpu.VMEM((bw,), jnp.float32),
            pltpu.VMEM((bw,), jnp.float32),
            pltpu.SemaphoreType.DMA,
        ],
    )
    def k(table_hbm, idx_hbm, out_hbm, idx_v, iidx_a, iidx_b, dat_a, dat_b,
          sem):
        wid = lax.axis_index("s") * info.num_cores + lax.axis_index("c")
        base = wid * bw
        pltpu.sync_copy(idx_hbm.at[pl.ds(base, bw)], idx_v)
        lane = lax.iota(jnp.int32, L)
        parity = lane & 1
        half = lane >> 1
        for c in range(2 * bw // L):
            nblk = idx_v[pl.ds((c // 2) * L, L)]
            nv = _vperm(nblk, half + 8 * (c % 2))
            val = nv * 2 + parity
            tgt = iidx_a if c < bw // L else iidx_b
            tgt[pl.ds((c % (bw // L)) * L, L)] = val
        h_a = pltpu.async_copy(table_hbm.at[iidx_a], dat_a, sem)
        h_b = pltpu.async_copy(table_hbm.at[iidx_b], dat_b, sem)
        h_a.wait()
        h_b.wait()
        scale = jnp.where(parity == 0, 90.0, 180.0)
        for j in range(bw // L):
            va = dat_a[pl.ds(j * L, L)]
            dat_a[pl.ds(j * L, L)] = jnp.minimum(jnp.maximum(va, -1.0), 1.0) * scale
            vb = dat_b[pl.ds(j * L, L)]
            dat_b[pl.ds(j * L, L)] = jnp.minimum(jnp.maximum(vb, -1.0), 1.0) * scale
        pltpu.sync_copy(dat_a, out_hbm.at[pl.ds(2 * base, bw)])
        pltpu.sync_copy(dat_b, out_hbm.at[pl.ds(2 * base + bw, bw)])

    return k(table_flat, n)


def kernel(img, coordinates, n):
    del img
    flat = _gather_gps(coordinates.reshape(-1), n)
    return flat.reshape(n.shape[0], 2)


# speedup vs baseline: 1.5941x; 1.0142x over previous
import functools

import jax
import jax.numpy as jnp
from jax import lax
from jax.experimental import pallas as pl
from jax.experimental.pallas import tpu as pltpu
from jax.experimental.pallas import tpu_sc as plsc

L = 16
G = 128

_PERM_DNUMS = lax.GatherDimensionNumbers(
    offset_dims=(), collapsed_slice_dims=(0,), start_index_map=(0,))


def _vperm(x, perm):
    return lax.gather(x, perm[:, None], _PERM_DNUMS, slice_sizes=(1,),
                      mode=lax.GatherScatterMode.PROMISE_IN_BOUNDS)


@jax.jit
def _gather_gps(table_flat, n):
    B = n.shape[0]
    info = plsc.get_sparse_core_info()
    num_workers = info.num_subcores
    bw = B // num_workers
    ng = 2 * bw // G
    mesh = plsc.VectorSubcoreMesh(
        core_axis_name="c", subcore_axis_name="s", num_cores=1)

    @functools.partial(
        pl.kernel,
        mesh=mesh,
        out_type=jax.ShapeDtypeStruct((2 * B,), jnp.float32),
        scratch_types=[
            pltpu.VMEM((bw,), jnp.int32),
            [pltpu.VMEM((G,), jnp.int32) for _ in range(ng)],
            [pltpu.VMEM((G,), jnp.float32) for _ in range(ng)],
            pltpu.SemaphoreType.DMA,
        ],
    )
    def k(table_hbm, idx_hbm, out_hbm, idx_v, iidx, dat, sem):
        wid = lax.axis_index("s")
        base = wid * bw
        pltpu.sync_copy(idx_hbm.at[pl.ds(base, bw)], idx_v)
        lane = lax.iota(jnp.int32, L)
        parity = lane & 1
        half = lane >> 1
        handles = []
        for g in range(ng):
            for c in range(G // L):
                blk = (g * (G // L) + c) // 2
                nblk = idx_v[pl.ds(blk * L, L)]
                nv = _vperm(nblk, half + 8 * (c % 2))
                iidx[g][pl.ds(c * L, L)] = nv * 2 + parity
            handles.append(pltpu.async_copy(table_hbm.at[iidx[g]], dat[g], sem))
        scale = jnp.where(parity == 0, 90.0, 180.0)
        for g in range(ng):
            handles[g].wait()
            for c in range(G // L):
                v = dat[g][pl.ds(c * L, L)]
                dat[g][pl.ds(c * L, L)] = (
                    jnp.minimum(jnp.maximum(v, -1.0), 1.0) * scale)
            pltpu.sync_copy(dat[g], out_hbm.at[pl.ds(2 * base + g * G, G)])

    return k(table_flat, n)


def kernel(img, coordinates, n):
    del img
    flat = _gather_gps(coordinates.reshape(-1), n)
    return flat.reshape(n.shape[0], 2)
